# baseline (speedup 1.0000x reference)
import functools

import jax
import jax.numpy as jnp
from jax import lax
from jax.experimental import pallas as pl
from jax.experimental.pallas import tpu as pltpu
from jax.experimental.pallas import tpu_sc as plsc

_NC = 2
_NS = 16
_NW = _NC * _NS

_HIST = 50
_D = 64
_PAIR = 2
_G = _PAIR * _HIST
_NBUF = 4


def _bag_body(left_hbm, right_hbm, table_hbm, out_l_hbm, out_r_hbm,
              idx_v, bufs, out_v, sems):
    ng = idx_v.shape[0]
    bw = out_v.shape[0]
    wid = lax.axis_index("s") * _NC + lax.axis_index("c")

    for side in range(2):
        names_hbm = left_hbm if side == 0 else right_hbm
        out_hbm = out_l_hbm if side == 0 else out_r_hbm

        pltpu.sync_copy(names_hbm.at[wid], idx_v)
        for b in range(_NBUF):
            pltpu.async_copy(table_hbm.at[idx_v.at[b]], bufs[b], sems[b])

        @pl.loop(0, ng, step=_NBUF)
        def _outer(g):
            for b in range(_NBUF):
                buf, sem = bufs[b], sems[b]
                j = g + b
                pltpu.make_async_copy(table_hbm.at[idx_v.at[j]], buf, sem).wait()
                for bag in range(_PAIR):
                    base = bag * _HIST

                    def _rb(r, acc, base=base, buf=buf):
                        return tuple(
                            acc[c] + buf[base + r, pl.ds(16 * c, 16)]
                            for c in range(4))

                    acc = tuple(buf[base, pl.ds(16 * c, 16)] for c in range(4))
                    acc = lax.fori_loop(1, _HIST, _rb, acc, unroll=7)
                    row = _PAIR * j + bag
                    for c in range(4):
                        out_v[row, pl.ds(16 * c, 16)] = acc[c]

                @pl.when(j + _NBUF < ng)
                def _(buf=buf, sem=sem, j=j):
                    pltpu.async_copy(table_hbm.at[idx_v.at[j + _NBUF]], buf, sem)

        pltpu.sync_copy(out_v, out_hbm.at[pl.ds(wid * bw, bw)])


def _embed_bags(left_idx, right_idx, table):
    nw, ng, g = left_idx.shape
    bw = ng * _PAIR
    b = nw * bw
    mesh = plsc.VectorSubcoreMesh(core_axis_name="c", subcore_axis_name="s")
    f = pl.kernel(
        _bag_body,
        out_type=(jax.ShapeDtypeStruct((b, _D), jnp.float32),
                  jax.ShapeDtypeStruct((b, _D), jnp.float32)),
        mesh=mesh,
        scratch_types=[
            pltpu.VMEM((ng, g), jnp.int32),
            [pltpu.VMEM((g, 2 * _D), jnp.float32) for _ in range(_NBUF)],
            pltpu.VMEM((bw, _D), jnp.float32),
            [pltpu.SemaphoreType.DMA for _ in range(_NBUF)],
        ],
        compiler_params=pltpu.CompilerParams(use_tc_tiling_on_sc=False),
    )
    return f(left_idx, right_idx, table)


def _mlp_body(xl_ref, xr_ref, w1_ref, b1_ref, w2_ref, b2_ref, out_ref):
    scale = 1.0 / _HIST
    x = jnp.concatenate((xl_ref[...] * scale, xr_ref[...] * scale), axis=1)
    h = lax.dot_general(x, w1_ref[...], (((1,), (1,)), ((), ())),
                        preferred_element_type=jnp.float32)
    h = jnp.maximum(h + b1_ref[...], 0.0)
    logits = lax.dot_general(h, w2_ref[...], (((1,), (1,)), ((), ())),
                             preferred_element_type=jnp.float32)
    logits = logits + b2_ref[...]
    m = jnp.max(logits, axis=1, keepdims=True)
    e = jnp.exp(logits - m)
    out_ref[...] = e / jnp.sum(e, axis=1, keepdims=True)


def _mlp(xl, xr, w1, b1, w2, b2):
    batch, d = xl.shape
    hidden, two_d = w1.shape
    ncls = w2.shape[0]
    bm = 4096
    grid = (batch // bm,)
    return pl.pallas_call(
        _mlp_body,
        grid=grid,
        in_specs=[
            pl.BlockSpec((bm, d), lambda i: (i, 0)),
            pl.BlockSpec((bm, d), lambda i: (i, 0)),
            pl.BlockSpec((hidden, two_d), lambda i: (0, 0)),
            pl.BlockSpec((1, hidden), lambda i: (0, 0)),
            pl.BlockSpec((ncls, hidden), lambda i: (0, 0)),
            pl.BlockSpec((1, ncls), lambda i: (0, 0)),
        ],
        out_specs=pl.BlockSpec((bm, ncls), lambda i: (i, 0)),
        out_shape=jax.ShapeDtypeStruct((batch, ncls), jnp.float32),
    )(xl, xr, w1, b1, w2, b2)


def kernel(left_names, right_names, emb_table, W1, b1, W2, b2):
    batch, hist = left_names.shape
    ng = batch // (_NW * _PAIR)
    li = left_names.reshape(_NW, ng, _G)
    ri = right_names.reshape(_NW, ng, _G)
    tpad = jnp.pad(emb_table, ((0, 0), (0, _D)))
    xl, xr = _embed_bags(li, ri, tpad)
    return _mlp(xl, xr, W1, b1.reshape(1, -1), W2, b2.reshape(1, -1))


# speedup vs baseline: 1.0061x; 1.0061x over previous
import functools

import jax
import jax.numpy as jnp
from jax import lax
from jax.experimental import pallas as pl
from jax.experimental.pallas import tpu as pltpu
from jax.experimental.pallas import tpu_sc as plsc

_NC = 2
_NS = 16
_NW = _NC * _NS

_HIST = 50
_D = 64
_PAIR = 2
_G = _PAIR * _HIST
_NBUF = 8


def _bag_body(left_hbm, right_hbm, table_hbm, out_l_hbm, out_r_hbm,
              idx_v, bufs, out_v, sems):
    ng = idx_v.shape[0]
    bw = out_v.shape[0]
    wid = lax.axis_index("s") * _NC + lax.axis_index("c")

    for side in range(2):
        names_hbm = left_hbm if side == 0 else right_hbm
        out_hbm = out_l_hbm if side == 0 else out_r_hbm

        pltpu.sync_copy(names_hbm.at[wid], idx_v)
        for b in range(_NBUF):
            pltpu.async_copy(table_hbm.at[idx_v.at[b]], bufs[b], sems[b])

        @pl.loop(0, ng, step=_NBUF)
        def _outer(g):
            for b in range(_NBUF):
                buf, sem = bufs[b], sems[b]
                j = g + b
                pltpu.make_async_copy(table_hbm.at[idx_v.at[j]], buf, sem).wait()
                for bag in range(_PAIR):
                    base = bag * _HIST

                    def _rb(r, acc, base=base, buf=buf):
                        return tuple(
                            acc[c] + buf[base + r, pl.ds(16 * c, 16)]
                            for c in range(4))

                    acc = tuple(buf[base, pl.ds(16 * c, 16)] for c in range(4))
                    acc = lax.fori_loop(1, _HIST, _rb, acc, unroll=14)
                    row = _PAIR * j + bag
                    for c in range(4):
                        out_v[row, pl.ds(16 * c, 16)] = acc[c]

                @pl.when(j + _NBUF < ng)
                def _(buf=buf, sem=sem, j=j):
                    pltpu.async_copy(table_hbm.at[idx_v.at[j + _NBUF]], buf, sem)

        pltpu.sync_copy(out_v, out_hbm.at[pl.ds(wid * bw, bw)])


def _embed_bags(left_idx, right_idx, table):
    nw, ng, g = left_idx.shape
    bw = ng * _PAIR
    b = nw * bw
    mesh = plsc.VectorSubcoreMesh(core_axis_name="c", subcore_axis_name="s")
    f = pl.kernel(
        _bag_body,
        out_type=(jax.ShapeDtypeStruct((b, _D), jnp.float32),
                  jax.ShapeDtypeStruct((b, _D), jnp.float32)),
        mesh=mesh,
        scratch_types=[
            pltpu.VMEM((ng, g), jnp.int32),
            [pltpu.VMEM((g, _D), jnp.float32) for _ in range(_NBUF)],
            pltpu.VMEM((bw, _D), jnp.float32),
            [pltpu.SemaphoreType.DMA for _ in range(_NBUF)],
        ],
        compiler_params=pltpu.CompilerParams(use_tc_tiling_on_sc=False),
    )
    return f(left_idx, right_idx, table)


def _mlp_body(xl_ref, xr_ref, w1_ref, b1_ref, w2_ref, b2_ref, out_ref):
    scale = 1.0 / _HIST
    x = jnp.concatenate((xl_ref[...] * scale, xr_ref[...] * scale), axis=1)
    h = lax.dot_general(x, w1_ref[...], (((1,), (1,)), ((), ())),
                        preferred_element_type=jnp.float32)
    h = jnp.maximum(h + b1_ref[...], 0.0)
    logits = lax.dot_general(h, w2_ref[...], (((1,), (1,)), ((), ())),
                             preferred_element_type=jnp.float32)
    logits = logits + b2_ref[...]
    m = jnp.max(logits, axis=1, keepdims=True)
    e = jnp.exp(logits - m)
    out_ref[...] = e / jnp.sum(e, axis=1, keepdims=True)


def _mlp(xl, xr, w1, b1, w2, b2):
    batch, d = xl.shape
    hidden, two_d = w1.shape
    ncls = w2.shape[0]
    bm = 4096
    grid = (batch // bm,)
    return pl.pallas_call(
        _mlp_body,
        grid=grid,
        in_specs=[
            pl.BlockSpec((bm, d), lambda i: (i, 0)),
            pl.BlockSpec((bm, d), lambda i: (i, 0)),
            pl.BlockSpec((hidden, two_d), lambda i: (0, 0)),
            pl.BlockSpec((1, hidden), lambda i: (0, 0)),
            pl.BlockSpec((ncls, hidden), lambda i: (0, 0)),
            pl.BlockSpec((1, ncls), lambda i: (0, 0)),
        ],
        out_specs=pl.BlockSpec((bm, ncls), lambda i: (i, 0)),
        out_shape=jax.ShapeDtypeStruct((batch, ncls), jnp.float32),
    )(xl, xr, w1, b1, w2, b2)


def kernel(left_names, right_names, emb_table, W1, b1, W2, b2):
    batch, hist = left_names.shape
    ng = batch // (_NW * _PAIR)
    li = left_names.reshape(_NW, ng, _G)
    ri = right_names.reshape(_NW, ng, _G)
    xl, xr = _embed_bags(li, ri, emb_table)
    return _mlp(xl, xr, W1, b1.reshape(1, -1), W2, b2.reshape(1, -1))


# speedup vs baseline: 1.1678x; 1.1607x over previous
import functools

import jax
import jax.numpy as jnp
from jax import lax
from jax.experimental import pallas as pl
from jax.experimental.pallas import tpu as pltpu
from jax.experimental.pallas import tpu_sc as plsc

_NC = 2
_NS = 16
_NW = _NC * _NS

_HIST = 50
_D = 64
_PAIR = 2
_G = _PAIR * _HIST
_NBUF = 8


def _bag_body(left_hbm, right_hbm, table_hbm, out_l_hbm, out_r_hbm,
              idx_v, bufs, out_v, sems):
    ng = idx_v.shape[0]
    bw = out_v.shape[0]
    wid = lax.axis_index("s") * _NC + lax.axis_index("c")

    for side in range(2):
        names_hbm = left_hbm if side == 0 else right_hbm
        out_hbm = out_l_hbm if side == 0 else out_r_hbm

        pltpu.sync_copy(names_hbm.at[wid], idx_v)
        for b in range(_NBUF):
            pltpu.async_copy(table_hbm.at[idx_v.at[b]], bufs[b], sems[b])

        @pl.loop(0, ng, step=_NBUF)
        def _outer(g):
            for b in range(_NBUF):
                buf, sem = bufs[b], sems[b]
                j = g + b
                pltpu.make_async_copy(table_hbm.at[idx_v.at[j]], buf, sem).wait()
                for bag in range(_PAIR):
                    base = bag * _HIST

                    def _rb(r, acc, base=base, buf=buf):
                        return tuple(
                            acc[c] + buf[base + r, pl.ds(16 * c, 16)]
                            for c in range(4))

                    acc = tuple(buf[base, pl.ds(16 * c, 16)] for c in range(4))
                    acc = lax.fori_loop(1, _HIST, _rb, acc, unroll=7)
                    row = _PAIR * j + bag
                    for c in range(4):
                        out_v[row, pl.ds(16 * c, 16)] = acc[c]

                @pl.when(j + _NBUF < ng)
                def _(buf=buf, sem=sem, j=j):
                    pltpu.async_copy(table_hbm.at[idx_v.at[j + _NBUF]], buf, sem)

        pltpu.sync_copy(out_v, out_hbm.at[pl.ds(wid * bw, bw)])


def _embed_bags(left_idx, right_idx, table):
    nw, ng, g = left_idx.shape
    bw = ng * _PAIR
    b = nw * bw
    mesh = plsc.VectorSubcoreMesh(core_axis_name="c", subcore_axis_name="s")
    f = pl.kernel(
        _bag_body,
        out_type=(jax.ShapeDtypeStruct((b, _D), jnp.float32),
                  jax.ShapeDtypeStruct((b, _D), jnp.float32)),
        mesh=mesh,
        scratch_types=[
            pltpu.VMEM((ng, g), jnp.int32),
            [pltpu.VMEM((g, _D), jnp.float32) for _ in range(_NBUF)],
            pltpu.VMEM((bw, _D), jnp.float32),
            [pltpu.SemaphoreType.DMA for _ in range(_NBUF)],
        ],
        compiler_params=pltpu.CompilerParams(use_tc_tiling_on_sc=False),
    )
    return f(left_idx, right_idx, table)


def _mlp_body(xl_ref, xr_ref, w1_ref, b1_ref, w2_ref, b2_ref, out_ref):
    scale = 1.0 / _HIST
    x = jnp.concatenate((xl_ref[...] * scale, xr_ref[...] * scale), axis=1)
    h = lax.dot_general(x, w1_ref[...], (((1,), (1,)), ((), ())),
                        preferred_element_type=jnp.float32)
    h = jnp.maximum(h + b1_ref[...], 0.0)
    logits = lax.dot_general(h, w2_ref[...], (((1,), (1,)), ((), ())),
                             preferred_element_type=jnp.float32)
    logits = logits + b2_ref[...]
    m = jnp.max(logits, axis=1, keepdims=True)
    e = jnp.exp(logits - m)
    out_ref[...] = e / jnp.sum(e, axis=1, keepdims=True)


def _mlp(xl, xr, w1, b1, w2, b2):
    batch, d = xl.shape
    hidden, two_d = w1.shape
    ncls = w2.shape[0]
    bm = 4096
    grid = (batch // bm,)
    return pl.pallas_call(
        _mlp_body,
        grid=grid,
        in_specs=[
            pl.BlockSpec((bm, d), lambda i: (i, 0)),
            pl.BlockSpec((bm, d), lambda i: (i, 0)),
            pl.BlockSpec((hidden, two_d), lambda i: (0, 0)),
            pl.BlockSpec((1, hidden), lambda i: (0, 0)),
            pl.BlockSpec((ncls, hidden), lambda i: (0, 0)),
            pl.BlockSpec((1, ncls), lambda i: (0, 0)),
        ],
        out_specs=pl.BlockSpec((bm, ncls), lambda i: (i, 0)),
        out_shape=jax.ShapeDtypeStruct((batch, ncls), jnp.float32),
    )(xl, xr, w1, b1, w2, b2)


def kernel(left_names, right_names, emb_table, W1, b1, W2, b2):
    batch, hist = left_names.shape
    ng = batch // (_NW * _PAIR)
    li = left_names.reshape(_NW, ng, _G)
    ri = right_names.reshape(_NW, ng, _G)
    xl, xr = _embed_bags(li, ri, emb_table)
    return _mlp(xl, xr, W1, b1.reshape(1, -1), W2, b2.reshape(1, -1))
